# baseline (speedup 1.0000x reference)
import jax
import jax.numpy as jnp
from jax import lax
from jax.experimental import pallas as pl
from jax.experimental.pallas import tpu as pltpu
from jax.experimental.pallas import tpu_sc as plsc
from jax._src.pallas.primitives import delay as _delay

N = 10000
E = 320000
IN = 128
HID = 128
R = 8

NROWS = 10240
NC = 2
NS = 16
NW = NC * NS
B = 128
EPT = 10240
E_PAD = NW * EPT
EROWS = E_PAD // B
ERPT = EPT // B
NBPC = 8
NCHUNK = ERPT // NBPC
ROWS_PER_TILE = NROWS // NW
ZROW = N


def _cnt_body(oneh_hbm, typm_hbm, doff_hbm, zeros_hbm, out_hbm,
              tb0, db0, rowbuf, cnt, gsem, isem):
    cid = lax.axis_index("c")
    sid = lax.axis_index("s")
    wid = cid * NS + sid
    erow_base = wid * ERPT
    slab = sid * NROWS

    def zrow(i, _z):
        pltpu.sync_copy(zeros_hbm, cnt.at[pl.ds(slab + i * B, B)])
        return _z

    lax.fori_loop(0, NROWS // B, zrow, None)

    def row_body(j, _c):
        ro = erow_base + j
        pltpu.async_copy(typm_hbm.at[ro], tb0, isem).wait()
        pltpu.async_copy(doff_hbm.at[ro], db0, isem).wait()
        pltpu.async_copy(oneh_hbm.at[tb0], rowbuf, gsem).wait()
        pltpu.sync_copy(rowbuf, cnt.at[db0], add=True)
        return _c

    lax.fori_loop(0, ERPT, row_body, None)
    pltpu.sync_copy(cnt.at[pl.ds(slab, NROWS)], out_hbm.at[wid])


@jax.jit
def _sc_counts(oneh, typm, doff):
    mesh = plsc.VectorSubcoreMesh(core_axis_name="c", subcore_axis_name="s")
    zeros2d = jnp.zeros((B, R), jnp.float32)
    f = pl.kernel(
        _cnt_body,
        out_type=jax.ShapeDtypeStruct((NW, NROWS, R), jnp.float32),
        mesh=mesh,
        scratch_types=[
            pltpu.VMEM((B,), jnp.int32),
            pltpu.VMEM((B,), jnp.int32),
            pltpu.VMEM((B, R), jnp.float32),
            pltpu.VMEM_SHARED((NS * NROWS, R), jnp.float32),
            pltpu.SemaphoreType.DMA,
            pltpu.SemaphoreType.DMA,
        ],
        compiler_params=pltpu.CompilerParams(use_tc_tiling_on_sc=False),
    )
    return f(oneh, typm, doff, zeros2d)


ZPT = NROWS // NS


def _sum_body(xp_hbm, midxr_hbm, dst_hbm, zeros_hbm, out_hbm,
              sb0, db0, rowbuf, acc, gsem, isem):
    cid = lax.axis_index("c")
    sid = lax.axis_index("s")
    wid = cid * NS + sid
    erow_base = wid * ERPT
    zbase = sid * ZPT

    def zrow(i, _z):
        pltpu.sync_copy(zeros_hbm, acc.at[pl.ds(zbase + i * B, B)])
        return _z

    lax.fori_loop(0, ZPT // B, zrow, None)
    plsc.subcore_barrier()

    def row_body(j, _c):
        ro = erow_base + j
        pltpu.async_copy(midxr_hbm.at[ro], sb0, isem).wait()
        pltpu.async_copy(dst_hbm.at[ro], db0, isem).wait()
        pltpu.async_copy(xp_hbm.at[sb0], rowbuf, gsem).wait()
        pltpu.sync_copy(rowbuf, acc.at[db0], add=True)
        return _c

    lax.fori_loop(0, ERPT, row_body, None)
    plsc.subcore_barrier()
    pltpu.sync_copy(acc.at[pl.ds(zbase, ZPT)],
                    out_hbm.at[cid, pl.ds(zbase, ZPT)])


@jax.jit
def _sc_segment_sums(xp, midx, dst2):
    mesh = plsc.VectorSubcoreMesh(core_axis_name="c", subcore_axis_name="s")
    zeros2d = jnp.zeros((B, IN), jnp.float32)
    f = pl.kernel(
        _sum_body,
        out_type=jax.ShapeDtypeStruct((NC, NROWS, IN), jnp.float32),
        mesh=mesh,
        scratch_types=[
            pltpu.VMEM((B,), jnp.int32),
            pltpu.VMEM((B,), jnp.int32),
            pltpu.VMEM((B, IN), jnp.float32),
            pltpu.VMEM_SHARED((NROWS, IN), jnp.float32),
            pltpu.SemaphoreType.DMA,
            pltpu.SemaphoreType.DMA,
        ],
    )
    outs = [f(xp, midx[r], dst2, zeros2d) for r in range(R)]
    return jnp.stack(outs)


def _prep_body(s_ref, t_ref, d_ref, o_ref, tm_ref, do_ref):
    r = pl.program_id(0)
    t = t_ref[...]
    o_ref[0] = jnp.where(t == r, s_ref[...], ZROW)
    tm_ref[...] = jnp.where(t < 0, R, t)
    row = jax.lax.broadcasted_iota(jnp.int32, (EROWS, B), 0)
    do_ref[...] = ((row // ERPT) % NS) * NROWS + d_ref[...]


@jax.jit
def _tc_prep(src2, typ2, dst2):
    return pl.pallas_call(
        _prep_body,
        grid=(R,),
        in_specs=[
            pl.BlockSpec((EROWS, B), lambda i: (0, 0)),
            pl.BlockSpec((EROWS, B), lambda i: (0, 0)),
            pl.BlockSpec((EROWS, B), lambda i: (0, 0)),
        ],
        out_specs=[
            pl.BlockSpec((1, EROWS, B), lambda i: (i, 0, 0)),
            pl.BlockSpec((EROWS, B), lambda i: (0, 0)),
            pl.BlockSpec((EROWS, B), lambda i: (0, 0)),
        ],
        out_shape=[
            jax.ShapeDtypeStruct((R, EROWS, B), jnp.int32),
            jax.ShapeDtypeStruct((EROWS, B), jnp.int32),
            jax.ShapeDtypeStruct((EROWS, B), jnp.int32),
        ],
    )(src2, typ2, dst2)


BN = 400


def _tc_body(x_ref, s_ref, c_ref, w_ref, root_ref, bias_ref, o_ref):
    xb = x_ref[...]
    acc = jnp.dot(xb, root_ref[...], preferred_element_type=jnp.float32)
    acc = acc + bias_ref[...]
    s = s_ref[...]
    cnt_all = jnp.sum(c_ref[...], axis=0)
    for r in range(R):
        t = s[2 * r] + s[2 * r + 1]
        h = t / jnp.maximum(cnt_all[:, r:r + 1], 1.0)
        acc = acc + jnp.dot(h, w_ref[r], preferred_element_type=jnp.float32)
    o_ref[...] = jnp.maximum(acc, 0.0)


@jax.jit
def _tc_combine(x, sums, cnts, weight, root, bias2d):
    grid = (N // BN,)
    return pl.pallas_call(
        _tc_body,
        grid=grid,
        in_specs=[
            pl.BlockSpec((BN, IN), lambda i: (i, 0)),
            pl.BlockSpec((R * NC, BN, IN), lambda i: (0, i, 0)),
            pl.BlockSpec((NW, BN, R), lambda i: (0, i, 0)),
            pl.BlockSpec((R, IN, HID), lambda i: (0, 0, 0)),
            pl.BlockSpec((IN, HID), lambda i: (0, 0)),
            pl.BlockSpec((1, HID), lambda i: (0, 0)),
        ],
        out_specs=pl.BlockSpec((BN, HID), lambda i: (i, 0)),
        out_shape=jax.ShapeDtypeStruct((N, HID), jnp.float32),
    )(x, sums, cnts, weight, root, bias2d)


def kernel(x, edge_index, edge_type, weight, root, bias):
    xp = jnp.zeros((NROWS, IN), jnp.float32).at[:N].set(x)
    src2 = jnp.pad(edge_index[0], (0, E_PAD - E)).reshape(EROWS, B)
    dst2 = jnp.pad(edge_index[1], (0, E_PAD - E)).reshape(EROWS, B)
    typ2 = jnp.pad(edge_type, (0, E_PAD - E),
                   constant_values=-1).reshape(EROWS, B)
    oneh = jnp.zeros((16, R), jnp.float32).at[jnp.arange(R),
                                              jnp.arange(R)].set(1.0)
    midx, typm, doff = _tc_prep(src2, typ2, dst2)
    cnts = _sc_counts(oneh, typm, doff)
    sums = _sc_segment_sums(xp, midx, dst2)
    sums = sums.reshape(R * NC, NROWS, IN)
    return _tc_combine(x, sums, cnts, weight, root, bias.reshape(1, HID))
